# baseline (device time: 214450 ns/iter reference)
import jax
import jax.numpy as jnp
from jax import lax
from jax.experimental import pallas as pl
from jax.experimental.pallas import tpu as pltpu


def _cast_bf16(w):
    e, a, b = w.shape

    def body(i_ref, o_ref):
        o_ref[...] = i_ref[...].astype(jnp.bfloat16)

    return pl.pallas_call(
        body,
        grid=(e,),
        out_shape=jax.ShapeDtypeStruct((e, a, b), jnp.bfloat16),
        in_specs=[pl.BlockSpec((1, a, b), lambda i: (i, 0, 0))],
        out_specs=pl.BlockSpec((1, a, b), lambda i: (i, 0, 0)),
    )(w)


def kernel(x, assign, W1, W2):
    T, D = x.shape
    E, _, F = W1.shape

    BT = 256
    NBT = T // BT
    NCH = 4
    CH = T // NCH
    NBL = CH // BT

    iota = lax.iota(jnp.int32, T)
    sa, perm = lax.sort((assign, iota), num_keys=1)
    _, inv = lax.sort((perm, iota), num_keys=1)
    meta = jnp.stack([perm, inv], axis=1)
    safb = sa.reshape(T, 1)
    xb = x.astype(jnp.bfloat16)
    W1b = _cast_bf16(W1)
    W2b = _cast_bf16(W2)

    def body(xb_ref, meta_ref, safb_ref, w1_ref, w2_ref, fin_ref,
             xs, xrem, aremb, orem, orecv,
             m_s, m_r, xc_s, xc_r, rc_s, rc_r):
        my_x = lax.axis_index("x")
        my_y = lax.axis_index("y")
        my_z = lax.axis_index("z")
        peer = (my_x, my_y, 1 - my_z)
        ebase = my_z * E

        barrier = pltpu.get_barrier_semaphore()
        pl.semaphore_signal(barrier, inc=1, device_id=peer,
                            device_id_type=pl.DeviceIdType.MESH)
        pl.semaphore_wait(barrier, 1)

        rdma_m = pltpu.make_async_remote_copy(
            src_ref=safb_ref, dst_ref=aremb, send_sem=m_s, recv_sem=m_r,
            device_id=peer, device_id_type=pl.DeviceIdType.MESH)
        rdma_m.start()

        xsends = []
        for c in range(NCH):
            def xsblk(b, carry):
                sl = pl.ds(b * BT, BT)
                pf = meta_ref[sl, 0:1]

                def kstep(kb, acc):
                    io = lax.broadcasted_iota(jnp.int32, (BT, BT), 1)
                    io = io + kb * BT
                    pb = (pf == io).astype(jnp.bfloat16)
                    return acc + jnp.dot(
                        pb, xb_ref[pl.ds(kb * BT, BT), :],
                        preferred_element_type=jnp.float32)

                acc = lax.fori_loop(0, NBT, kstep,
                                    jnp.zeros((BT, D), jnp.float32))
                xs[sl, :] = acc.astype(jnp.bfloat16)
                return carry

            lax.fori_loop(c * NBL, (c + 1) * NBL, xsblk, 0)
            r = pltpu.make_async_remote_copy(
                src_ref=xs.at[pl.ds(c * CH, CH), :],
                dst_ref=xrem.at[pl.ds(c * CH, CH), :],
                send_sem=xc_s.at[c], recv_sem=xc_r.at[c],
                device_id=peer, device_id_type=pl.DeviceIdType.MESH)
            r.start()
            xsends.append(r)

        def lblk(b, carry):
            sl = pl.ds(b * BT, BT)
            xv = xs[sl, :]
            ab = safb_ref[sl, :]
            fin_ref[sl, :] = jnp.zeros((BT, D), jnp.bfloat16)
            for e in range(E):
                egb = ebase + e

                @pl.when(jnp.any(ab == egb))
                def _():
                    h = jnp.dot(xv, w1_ref[e],
                                preferred_element_type=jnp.float32)
                    h = jnp.maximum(h, 0.0).astype(jnp.bfloat16)
                    y = jnp.dot(h, w2_ref[e],
                                preferred_element_type=jnp.float32)
                    fin_ref[sl, :] = fin_ref[sl, :] + jnp.where(
                        ab == egb, y, 0.0).astype(jnp.bfloat16)
            return carry

        lax.fori_loop(0, NBT, lblk, 0)

        rdma_m.wait()

        rsends = []
        for c in range(NCH):
            xsends[c].wait()
            if c >= 2:
                rsends[c - 2].wait_send()
            slot = c % 2

            def rblk(b2, carry):
                slg = pl.ds(c * CH + b2 * BT, BT)
                sll = pl.ds(b2 * BT, BT)
                xv = xrem[slg, :]
                ab = aremb[slg, :]
                orem[slot, sll, :] = jnp.zeros((BT, D), jnp.bfloat16)
                for e in range(E):
                    egb = ebase + e

                    @pl.when(jnp.any(ab == egb))
                    def _():
                        h = jnp.dot(xv, w1_ref[e],
                                    preferred_element_type=jnp.float32)
                        h = jnp.maximum(h, 0.0).astype(jnp.bfloat16)
                        y = jnp.dot(h, w2_ref[e],
                                    preferred_element_type=jnp.float32)
                        orem[slot, sll, :] = orem[slot, sll, :] + jnp.where(
                            ab == egb, y, 0.0).astype(jnp.bfloat16)
                return carry

            lax.fori_loop(0, NBL, rblk, 0)
            r = pltpu.make_async_remote_copy(
                src_ref=orem.at[slot],
                dst_ref=orecv.at[pl.ds(c * CH, CH), :],
                send_sem=rc_s.at[c], recv_sem=rc_r.at[c],
                device_id=peer, device_id_type=pl.DeviceIdType.MESH)
            r.start()
            rsends.append(r)

        for c in range(NCH):
            rsends[c].wait_recv()
            sl = pl.ds(c * CH, CH)
            fin_ref[sl, :] = fin_ref[sl, :] + orecv[sl, :]
        rsends[NCH - 2].wait_send()
        rsends[NCH - 1].wait_send()

        def ublk(b, carry):
            sl = pl.ds(b * BT, BT)
            qf = meta_ref[sl, 1:2]

            def kstep(kb, acc):
                io = lax.broadcasted_iota(jnp.int32, (BT, BT), 1)
                io = io + kb * BT
                pb = (qf == io).astype(jnp.bfloat16)
                return acc + jnp.dot(
                    pb, fin_ref[pl.ds(kb * BT, BT), :],
                    preferred_element_type=jnp.float32)

            acc = lax.fori_loop(0, NBT, kstep,
                                jnp.zeros((BT, D), jnp.float32))
            xs[sl, :] = acc.astype(jnp.bfloat16)
            return carry

        lax.fori_loop(0, NBT, ublk, 0)
        fin_ref[...] = xs[...]

    fin = pl.pallas_call(
        body,
        out_shape=jax.ShapeDtypeStruct((T, D), jnp.bfloat16),
        in_specs=[
            pl.BlockSpec(memory_space=pltpu.VMEM),
            pl.BlockSpec(memory_space=pltpu.VMEM),
            pl.BlockSpec(memory_space=pltpu.VMEM),
            pl.BlockSpec(memory_space=pltpu.VMEM),
            pl.BlockSpec(memory_space=pltpu.VMEM),
        ],
        out_specs=pl.BlockSpec(memory_space=pltpu.VMEM),
        scratch_shapes=[
            pltpu.VMEM((T, D), jnp.bfloat16),
            pltpu.VMEM((T, D), jnp.bfloat16),
            pltpu.VMEM((T, 1), jnp.int32),
            pltpu.VMEM((2, CH, D), jnp.bfloat16),
            pltpu.VMEM((T, D), jnp.bfloat16),
            pltpu.SemaphoreType.DMA,
            pltpu.SemaphoreType.DMA,
            pltpu.SemaphoreType.DMA((NCH,)),
            pltpu.SemaphoreType.DMA((NCH,)),
            pltpu.SemaphoreType.DMA((NCH,)),
            pltpu.SemaphoreType.DMA((NCH,)),
        ],
        compiler_params=pltpu.CompilerParams(
            collective_id=0, vmem_limit_bytes=110 * 1024 * 1024
        ),
    )(xb, meta, safb, W1b, W2b)

    return fin.astype(jnp.float32)
